# initial kernel scaffold (unmeasured)
import jax
import jax.numpy as jnp
from jax import lax
from jax.experimental import pallas as pl
from jax.experimental.pallas import tpu as pltpu

N_DEV = 8


def _gelu(y):
    c = 0.7978845608028654
    return 0.5 * y * (1.0 + jnp.tanh(c * (y + 0.044715 * y * y * y)))


def kernel(x, w_mat):
    m_sh, k = x.shape
    _, n = w_mat.shape
    bn = n // N_DEV
    m_tot = m_sh * N_DEV

    def body(x_ref, w_hbm, out_ref, xb_ref, w_bufs, y_blocks, comm_ref,
             w_sems, send_sems, recv_sems):
        my = lax.axis_index("i")

        barrier = pltpu.get_barrier_semaphore()
        for p in range(N_DEV):
            pl.semaphore_signal(
                barrier, inc=1,
                device_id=(p,), device_id_type=pl.DeviceIdType.MESH,
            )
        pl.semaphore_wait(barrier, N_DEV)

        xb_ref[...] = x_ref[...].astype(jnp.bfloat16)

        def w_copy(t, slot):
            blk = (my + t) % N_DEV
            return pltpu.make_async_copy(
                w_hbm.at[:, pl.ds(blk * bn, bn)],
                w_bufs.at[slot],
                w_sems.at[slot],
            )

        def send_rdma(t):
            tgt = (my + t) % N_DEV
            return pltpu.make_async_remote_copy(
                src_ref=y_blocks.at[t],
                dst_ref=comm_ref.at[my],
                send_sem=send_sems.at[t],
                recv_sem=recv_sems.at[my],
                device_id=(tgt,),
                device_id_type=pl.DeviceIdType.MESH,
            )

        w_copy(0, 0).start()
        for t in range(N_DEV):
            slot = t % 2
            if t + 1 < N_DEV:
                w_copy(t + 1, 1 - slot).start()
            w_copy(t, slot).wait()
            wb = w_bufs[slot].astype(jnp.bfloat16)
            yf = lax.dot_general(
                xb_ref[...], wb, (((1,), (0,)), ((), ())),
                preferred_element_type=jnp.float32,
            )
            yf = _gelu(yf)
            if t == 0:
                out_ref[pl.ds(my * m_sh, m_sh), :] = yf
            else:
                y_blocks[t, :, :] = yf.astype(jnp.bfloat16)
                send_rdma(t).start()

        for src in range(N_DEV):
            @pl.when(src != my)
            def _():
                pltpu.make_async_remote_copy(
                    src_ref=y_blocks.at[1],
                    dst_ref=comm_ref.at[src],
                    send_sem=send_sems.at[1],
                    recv_sem=recv_sems.at[src],
                    device_id=(src,),
                    device_id_type=pl.DeviceIdType.MESH,
                ).wait_recv()
                out_ref[pl.ds(src * m_sh, m_sh), :] = (
                    comm_ref[src].astype(jnp.float32)
                )

        for t in range(1, N_DEV):
            send_rdma(t).wait_send()

    return pl.pallas_call(
        body,
        out_shape=jax.ShapeDtypeStruct((m_tot, bn), jnp.float32),
        in_specs=[
            pl.BlockSpec(memory_space=pltpu.VMEM),
            pl.BlockSpec(memory_space=pltpu.ANY),
        ],
        out_specs=pl.BlockSpec(memory_space=pltpu.VMEM),
        scratch_shapes=[
            pltpu.VMEM((m_sh, k), jnp.bfloat16),
            pltpu.VMEM((2, k, bn), jnp.float32),
            pltpu.VMEM((N_DEV, m_sh, bn), jnp.bfloat16),
            pltpu.VMEM((N_DEV, m_sh, bn), jnp.bfloat16),
            pltpu.SemaphoreType.DMA((2,)),
            pltpu.SemaphoreType.DMA((N_DEV,)),
            pltpu.SemaphoreType.DMA((N_DEV,)),
        ],
        compiler_params=pltpu.CompilerParams(collective_id=0),
    )(x, w_mat)


# baseline (device time: 127616 ns/iter reference)
import jax
import jax.numpy as jnp
from jax import lax
from jax.experimental import pallas as pl
from jax.experimental.pallas import tpu as pltpu

N_DEV = 8


def _gelu(y):
    c = 0.7978845608028654
    return 0.5 * y * (1.0 + jnp.tanh(c * (y + 0.044715 * y * y * y)))


def kernel(x, w_mat):
    m_sh, k = x.shape
    _, n = w_mat.shape
    bn = n // N_DEV
    m_tot = m_sh * N_DEV

    def body(x_ref, w_hbm, out_ref, xb_ref, w_bufs, y_blocks, comm_ref,
             w_sems, send_sems, recv_sems):
        my = lax.axis_index("i")

        barrier = pltpu.get_barrier_semaphore()
        for p in range(N_DEV):
            pl.semaphore_signal(
                barrier, inc=1,
                device_id=(p,), device_id_type=pl.DeviceIdType.MESH,
            )
        pl.semaphore_wait(barrier, N_DEV)

        xb_ref[...] = x_ref[...].astype(jnp.bfloat16)

        n_sub = 4
        bh = bn // n_sub

        def w_copy(s, slot):
            t, sub = divmod(s, n_sub)
            blk = (my + t) % N_DEV
            return pltpu.make_async_copy(
                w_hbm.at[:, pl.ds(blk * bn + sub * bh, bh)],
                w_bufs.at[slot],
                w_sems.at[slot],
            )

        def send_rdma(t):
            tgt = (my + t) % N_DEV
            return pltpu.make_async_remote_copy(
                src_ref=y_blocks.at[t],
                dst_ref=comm_ref.at[my],
                send_sem=send_sems.at[t],
                recv_sem=recv_sems.at[my],
                device_id=(tgt,),
                device_id_type=pl.DeviceIdType.MESH,
            )

        w_copy(0, 0).start()
        for s in range(n_sub * N_DEV):
            t, sub = divmod(s, n_sub)
            slot = s % 2
            if s + 1 < n_sub * N_DEV:
                w_copy(s + 1, 1 - slot).start()
            w_copy(s, slot).wait()
            wb = w_bufs[slot].astype(jnp.bfloat16)
            yf = lax.dot_general(
                xb_ref[...], wb, (((1,), (0,)), ((), ())),
                preferred_element_type=jnp.float32,
            )
            yf = _gelu(yf)
            if t == 0:
                out_ref[pl.ds(my * m_sh, m_sh), sub * bh:(sub + 1) * bh] = yf
            else:
                y_blocks[t, :, sub * bh:(sub + 1) * bh] = yf.astype(jnp.bfloat16)
                if sub == n_sub - 1:
                    send_rdma(t).start()

        for src in range(N_DEV):
            @pl.when(src != my)
            def _():
                pltpu.make_async_remote_copy(
                    src_ref=y_blocks.at[1],
                    dst_ref=comm_ref.at[src],
                    send_sem=send_sems.at[1],
                    recv_sem=recv_sems.at[src],
                    device_id=(src,),
                    device_id_type=pl.DeviceIdType.MESH,
                ).wait_recv()
                out_ref[pl.ds(src * m_sh, m_sh), :] = (
                    comm_ref[src].astype(jnp.float32)
                )

        for t in range(1, N_DEV):
            send_rdma(t).wait_send()

    return pl.pallas_call(
        body,
        out_shape=jax.ShapeDtypeStruct((m_tot, bn), jnp.float32),
        in_specs=[
            pl.BlockSpec(memory_space=pltpu.VMEM),
            pl.BlockSpec(memory_space=pl.ANY),
        ],
        out_specs=pl.BlockSpec(memory_space=pltpu.VMEM),
        scratch_shapes=[
            pltpu.VMEM((m_sh, k), jnp.bfloat16),
            pltpu.VMEM((2, k, bn // 4), jnp.float32),
            pltpu.VMEM((N_DEV, m_sh, bn), jnp.bfloat16),
            pltpu.VMEM((N_DEV, m_sh, bn), jnp.bfloat16),
            pltpu.SemaphoreType.DMA((2,)),
            pltpu.SemaphoreType.DMA((N_DEV,)),
            pltpu.SemaphoreType.DMA((N_DEV,)),
        ],
        compiler_params=pltpu.CompilerParams(
            collective_id=0,
            vmem_limit_bytes=63 * 1024 * 1024,
        ),
    )(x, w_mat)
